# Initial kernel scaffold; baseline (speedup 1.0000x reference)
import jax
import jax.numpy as jnp
from jax.experimental import pallas as pl


def kernel(x, gate_w, wg, wu, wd):
    raise NotImplementedError("write your pallas kernel here")


# speedup vs baseline: 1.5811x; 1.5811x over previous
import functools
import jax
import jax.numpy as jnp
from jax import lax
from jax.experimental import pallas as pl
from jax.experimental.pallas import tpu as pltpu

TM = 256


def _router_schedule_body(xf_ref, gw_ref, l128_ref, l32_ref, sl8_ref,
                          logits_ref, w_ref, pos_ref, off2_ref):
    S, Dm = xf_ref.shape
    Ee = gw_ref.shape[0]
    xfv = xf_ref[...]
    logits = jnp.dot(xfv, gw_ref[...].T, preferred_element_type=jnp.float32)
    logits_ref[...] = logits

    m = jnp.max(logits, axis=-1, keepdims=True)
    ex = jnp.exp(logits - m)
    rw = ex / jnp.sum(ex, axis=-1, keepdims=True)

    iota_e = lax.broadcasted_iota(jnp.int32, (S, Ee), 1)
    m1 = jnp.max(rw, axis=-1, keepdims=True)
    e1 = jnp.min(jnp.where(rw == m1, iota_e, Ee), axis=-1, keepdims=True)
    oh1 = iota_e == e1
    rwm = jnp.where(oh1, -1.0, rw)
    m2 = jnp.max(rwm, axis=-1, keepdims=True)
    e2 = jnp.min(jnp.where(rwm == m2, iota_e, Ee), axis=-1, keepdims=True)
    oh2 = iota_e == e2
    denom = m1 + m2
    w_ref[...] = jnp.concatenate([m1 / denom, m2 / denom], axis=1)

    O = jnp.concatenate([oh1, oh2], axis=0).astype(jnp.float32)
    cnt = jnp.sum(O, axis=0, keepdims=True)
    pc = jnp.ceil(cnt / TM) * TM
    off = jnp.dot(pc, sl8_ref[...], preferred_element_type=jnp.float32)
    off2_ref[...] = (off + pc).astype(jnp.int32)

    NA = 2 * S
    NB = NA // 128
    l128 = l128_ref[...]
    blocks = [O[i * 128:(i + 1) * 128, :] for i in range(NB)]
    s_rows = [jnp.sum(b, axis=0, keepdims=True) for b in blocks]
    sblk = jnp.concatenate(s_rows, axis=0)
    base = jnp.dot(l32_ref[...], sblk, preferred_element_type=jnp.float32)
    pos_parts = []
    for i in range(NB):
        r = jnp.dot(l128, blocks[i], preferred_element_type=jnp.float32)
        r = r + base[i:i + 1, :] + off
        pos_parts.append(jnp.sum(blocks[i] * r, axis=1, keepdims=True))
    pos_ref[...] = jnp.concatenate(pos_parts, axis=0).astype(jnp.int32)


def _gmm_body(te_ref, xs_ref, wg_ref, wu_ref, wd_ref, ys_ref):
    xb = xs_ref[...]
    a1 = lax.dot_general(xb, wg_ref[0], (((1,), (1,)), ((), ())),
                         preferred_element_type=jnp.float32)
    a2 = lax.dot_general(xb, wu_ref[0], (((1,), (1,)), ((), ())),
                         preferred_element_type=jnp.float32)
    h = a1 * jax.nn.sigmoid(a1) * a2
    ys_ref[...] = lax.dot_general(h, wd_ref[0], (((1,), (1,)), ((), ())),
                                  preferred_element_type=jnp.float32)


def _run_router(xf, gate_w):
    S, Dm = xf.shape
    Ee = gate_w.shape[0]
    NA = 2 * S
    NB = NA // 128
    l128 = jnp.tril(jnp.ones((128, 128), jnp.float32), -1)
    l32 = jnp.tril(jnp.ones((NB, NB), jnp.float32), -1)
    sl8 = jnp.triu(jnp.ones((Ee, Ee), jnp.float32), 1)
    return pl.pallas_call(
        _router_schedule_body,
        out_shape=(
            jax.ShapeDtypeStruct((S, Ee), jnp.float32),
            jax.ShapeDtypeStruct((S, 2), jnp.float32),
            jax.ShapeDtypeStruct((NA, 1), jnp.int32),
            jax.ShapeDtypeStruct((1, Ee), jnp.int32),
        ),
    )(xf, gate_w, l128, l32, sl8)


def _run_gmm(xs, wg, wu, wd, tile_expert, nt):
    P, Dm = xs.shape
    Ee, FF, _ = wg.shape
    grid_spec = pltpu.PrefetchScalarGridSpec(
        num_scalar_prefetch=1,
        grid=(nt,),
        in_specs=[
            pl.BlockSpec((TM, Dm), lambda g, te: (g, 0)),
            pl.BlockSpec((1, FF, Dm), lambda g, te: (te[g], 0, 0)),
            pl.BlockSpec((1, FF, Dm), lambda g, te: (te[g], 0, 0)),
            pl.BlockSpec((1, Dm, FF), lambda g, te: (te[g], 0, 0)),
        ],
        out_specs=pl.BlockSpec((TM, Dm), lambda g, te: (g, 0)),
    )
    return pl.pallas_call(
        _gmm_body,
        grid_spec=grid_spec,
        out_shape=jax.ShapeDtypeStruct((P, Dm), jnp.float32),
        compiler_params=pltpu.CompilerParams(
            dimension_semantics=("arbitrary",)),
    )(tile_expert, xs, wg, wu, wd)


def kernel(x, gate_w, wg, wu, wd):
    b, s, d = x.shape
    Ee = gate_w.shape[0]
    xf = x.reshape(b * s, d)
    S = b * s
    NA = 2 * S
    P = NA + Ee * TM
    NT = P // TM

    logits, w, pos, off2 = _run_router(xf, gate_w)
    pos = pos.reshape(NA)

    tiles = jnp.arange(NT, dtype=jnp.int32) * TM
    te = jnp.minimum(Ee - 1,
                     jnp.sum((tiles[:, None] >= off2[0][None, :]).astype(
                         jnp.int32), axis=1))

    src = jnp.concatenate([jnp.arange(S), jnp.arange(S)])
    inv = jnp.zeros((P,), jnp.int32).at[pos].set(src.astype(jnp.int32))
    xs = xf[inv]

    ys = _run_gmm(xs, wg, wu, wd, te, NT)

    y0 = ys[pos[:S]]
    y1 = ys[pos[S:]]
    final = w[:, 0:1] * y0 + w[:, 1:2] * y1
    return final.reshape(b, s, d), logits
